# baseline (speedup 1.0000x reference)
import functools

import jax
import jax.numpy as jnp
from jax import lax
from jax.experimental import pallas as pl
from jax.experimental.pallas import tpu as pltpu
from jax.experimental.pallas import tpu_sc as plsc

BATCH = 16384
HIDDEN = 1024
CHUNK = 32
N_BUF = 2


@jax.jit
def _embed(labels, table):
    info = plsc.get_sparse_core_info()
    num_workers = info.num_cores * info.num_subcores
    b_per_w = BATCH // num_workers
    n_chunks = b_per_w // CHUNK

    mesh = plsc.VectorSubcoreMesh(core_axis_name="c", subcore_axis_name="s")

    @functools.partial(
        pl.kernel,
        mesh=mesh,
        out_type=jax.ShapeDtypeStruct((BATCH, HIDDEN), jnp.float32),
        scratch_types=[
            pltpu.VMEM((b_per_w,), jnp.int32),
            pltpu.VMEM((N_BUF, CHUNK, HIDDEN), jnp.float32),
            pltpu.SemaphoreType.DMA((N_BUF,)),
            pltpu.SemaphoreType.DMA((N_BUF,)),
        ],
    )
    def k(labels_hbm, table_hbm, out_hbm, idx_v, rows_v, gsem, ssem):
        wid = lax.axis_index("s") * info.num_cores + lax.axis_index("c")
        base = wid * b_per_w
        pltpu.sync_copy(labels_hbm.at[pl.ds(base, b_per_w)], idx_v)

        def start_gather(i):
            slot = i % N_BUF
            return pltpu.async_copy(
                table_hbm.at[idx_v.at[pl.ds(i * CHUNK, CHUNK)]],
                rows_v.at[slot],
                gsem.at[slot],
            )

        def start_scatter(i):
            slot = i % N_BUF
            return pltpu.async_copy(
                rows_v.at[slot],
                out_hbm.at[pl.ds(base + i * CHUNK, CHUNK)],
                ssem.at[slot],
            )

        g = [None] * n_chunks
        s = [None] * n_chunks
        g[0] = start_gather(0)
        for i in range(n_chunks):
            if i + 1 < n_chunks:
                if i >= 1:
                    s[i - 1].wait()
                g[i + 1] = start_gather(i + 1)
            g[i].wait()
            s[i] = start_scatter(i)
        s[n_chunks - 2].wait()
        s[n_chunks - 1].wait()

    return k(labels, table)


def kernel(labels, table):
    return _embed(labels, table)
